# baseline (speedup 1.0000x reference)
import functools

import jax
import jax.numpy as jnp
from jax import lax
from jax.experimental import pallas as pl
from jax.experimental.pallas import tpu as pltpu
from jax.experimental.pallas import tpu_sc as plsc

BATCH = 1024
SEQ = 200
VOCAB = 1000000
DIM = 64
LANES = 16
NUM_CORES = 2
NUM_SUBCORES = 16
NW = NUM_CORES * NUM_SUBCORES
BT = BATCH // 128
NUNITS = SEQ * BT
UNITS_PER_W = NUNITS // NW
NBUF = 5
NOUT = UNITS_PER_W // NBUF

TW = 512
RGRID = (VOCAB + TW - 1) // TW


def _relayout_body(src_ref, dst_ref):
    eye = (lax.broadcasted_iota(jnp.int32, (DIM, DIM), 0) ==
           lax.broadcasted_iota(jnp.int32, (DIM, DIM), 1)
           ).astype(jnp.float32)
    dst_ref[:, 0:DIM] = lax.dot_general(
        src_ref[...], eye, (((0,), (0,)), ((), ())),
        precision=lax.Precision.HIGHEST,
        preferred_element_type=jnp.float32)


def _sc_body(idx_hbm, tok_hbm, pos_hbm, out_hbm,
             ibufs, pos_v, gbufs, obufs, gsems, wsems):
    wid = lax.axis_index("s") * NUM_CORES + lax.axis_index("c")
    ubase = wid * UNITS_PER_W
    lmin = lax.min(ubase // BT, SEQ - 8)
    pltpu.sync_copy(pos_hbm.at[pl.ds(lmin, 8)], pos_v)

    def stage_and_fire(u, b):
        l = u // BT
        bt = u % BT
        pltpu.sync_copy(idx_hbm.at[l // 8, bt, l % 8], ibufs[b])
        return pltpu.async_copy(tok_hbm.at[ibufs[b]], gbufs[b], gsems[b])

    for b in range(NBUF):
        stage_and_fire(ubase + b, b)

    def outer(o, carry):
        for b in range(NBUF):
            u = ubase + o * NBUF + b
            l = u // BT
            bt = u % BT
            pltpu.make_async_copy(tok_hbm.at[ibufs[b]], gbufs[b],
                                  gsems[b]).wait()

            @pl.when(o > 0)
            def _drain():
                pltpu.make_async_copy(obufs[b], out_hbm.at[bt, :, l],
                                      wsems[b]).wait()

            pvecs = [pos_v[l - lmin, pl.ds(k * LANES, LANES)]
                     for k in range(DIM // LANES)]

            def tbody(t, c2, _b=b, _pv=pvecs):
                for k in range(DIM // LANES):
                    sl = pl.ds(k * LANES, LANES)
                    obufs[_b][t, sl] = gbufs[_b][t, sl] + _pv[k]
                return c2

            lax.fori_loop(0, 128, tbody, 0, unroll=4)

            pltpu.async_copy(obufs[b], out_hbm.at[bt, :, l], wsems[b])

            @pl.when(o < NOUT - 1)
            def _next():
                stage_and_fire(u + NBUF, b)
        return carry

    lax.fori_loop(0, NOUT, outer, 0)

    for b in range(NBUF):
        u = ubase + (NOUT - 1) * NBUF + b
        l = u // BT
        bt = u % BT
        pltpu.make_async_copy(obufs[b], out_hbm.at[bt, :, l],
                              wsems[b]).wait()


@jax.jit
def _run(idx4, tokT, pos):
    tokP = pl.pallas_call(
        _relayout_body,
        grid=(RGRID,),
        in_specs=[pl.BlockSpec((DIM, TW), lambda i: (0, i))],
        out_specs=pl.BlockSpec((TW, 128), lambda i: (i, 0)),
        out_shape=jax.ShapeDtypeStruct((VOCAB, 128), jnp.float32),
    )(tokT)
    tok2 = tokP.reshape(2 * VOCAB, DIM)

    mesh = plsc.VectorSubcoreMesh(core_axis_name="c", subcore_axis_name="s")
    f = functools.partial(
        pl.kernel,
        out_type=jax.ShapeDtypeStruct((BT, 128, SEQ, DIM), jnp.float32),
        mesh=mesh,
        scratch_types=[
            [pltpu.VMEM((128,), jnp.int32)] * NBUF,
            pltpu.VMEM((8, DIM), jnp.float32),
            [pltpu.VMEM((128, DIM), jnp.float32)] * NBUF,
            [pltpu.VMEM((128, DIM), jnp.float32)] * NBUF,
            [pltpu.SemaphoreType.DMA] * NBUF,
            [pltpu.SemaphoreType.DMA] * NBUF,
        ],
        compiler_params=pltpu.CompilerParams(use_tc_tiling_on_sc=False,
                                             needs_layout_passes=False),
    )(_sc_body)
    out4 = f(idx4, tok2, pos)
    return out4.reshape(BATCH, SEQ, DIM)


def kernel(inputs, token_table, position_table):
    idx4 = (inputs.astype(jnp.int32)
            .reshape(BT, 128, SEQ // 8, 8)
            .transpose(2, 0, 3, 1)) * 2
    return _run(idx4, token_table.T, position_table)


# speedup vs baseline: 1.1399x; 1.1399x over previous
import functools

import jax
import jax.numpy as jnp
from jax import lax
from jax.experimental import pallas as pl
from jax.experimental.pallas import tpu as pltpu
from jax.experimental.pallas import tpu_sc as plsc

BATCH = 1024
SEQ = 200
VOCAB = 1000000
DIM = 64
LANES = 16
NUM_CORES = 2
NUM_SUBCORES = 16
NW = NUM_CORES * NUM_SUBCORES
BT = BATCH // 128
NUNITS = SEQ * BT
UNITS_PER_W = NUNITS // NW
NBUF = 5
NOUT = UNITS_PER_W // NBUF

TW = 512
RGRID = (VOCAB + TW - 1) // TW


def _relayout_body(src_ref, dst_ref):
    dst_ref[:, 0:DIM] = jnp.transpose(src_ref[...], (1, 0))


def _sc_body(idx_hbm, tok_hbm, pos_hbm, out_hbm,
             ibufs, pos_v, gbufs, obufs, gsems, wsems):
    wid = lax.axis_index("s") * NUM_CORES + lax.axis_index("c")
    ubase = wid * UNITS_PER_W
    lmin = lax.min(ubase // BT, SEQ - 8)
    pltpu.sync_copy(pos_hbm.at[pl.ds(lmin, 8)], pos_v)

    def stage_and_fire(u, b):
        l = u // BT
        bt = u % BT
        pltpu.sync_copy(idx_hbm.at[l // 8, bt, l % 8], ibufs[b])
        return pltpu.async_copy(tok_hbm.at[ibufs[b]], gbufs[b], gsems[b])

    for b in range(NBUF):
        stage_and_fire(ubase + b, b)

    def outer(o, carry):
        for b in range(NBUF):
            u = ubase + o * NBUF + b
            l = u // BT
            bt = u % BT
            pltpu.make_async_copy(tok_hbm.at[ibufs[b]], gbufs[b],
                                  gsems[b]).wait()

            @pl.when(o > 0)
            def _drain():
                pltpu.make_async_copy(obufs[b], out_hbm.at[bt, :, l],
                                      wsems[b]).wait()

            pvecs = [pos_v[l - lmin, pl.ds(k * LANES, LANES)]
                     for k in range(DIM // LANES)]

            def tbody(t, c2, _b=b, _pv=pvecs):
                for k in range(DIM // LANES):
                    sl = pl.ds(k * LANES, LANES)
                    obufs[_b][t, sl] = gbufs[_b][t, sl] + _pv[k]
                return c2

            lax.fori_loop(0, 128, tbody, 0, unroll=4)

            pltpu.async_copy(obufs[b], out_hbm.at[bt, :, l], wsems[b])

            @pl.when(o < NOUT - 1)
            def _next():
                stage_and_fire(u + NBUF, b)
        return carry

    lax.fori_loop(0, NOUT, outer, 0)

    for b in range(NBUF):
        u = ubase + (NOUT - 1) * NBUF + b
        l = u // BT
        bt = u % BT
        pltpu.make_async_copy(obufs[b], out_hbm.at[bt, :, l],
                              wsems[b]).wait()


@jax.jit
def _run(idx4, tokT, pos):
    tokP = pl.pallas_call(
        _relayout_body,
        grid=(RGRID,),
        in_specs=[pl.BlockSpec((DIM, TW), lambda i: (0, i))],
        out_specs=pl.BlockSpec((TW, 128), lambda i: (i, 0)),
        out_shape=jax.ShapeDtypeStruct((VOCAB, 128), jnp.float32),
    )(tokT)
    tok2 = tokP.reshape(2 * VOCAB, DIM)

    mesh = plsc.VectorSubcoreMesh(core_axis_name="c", subcore_axis_name="s")
    f = functools.partial(
        pl.kernel,
        out_type=jax.ShapeDtypeStruct((BT, 128, SEQ, DIM), jnp.float32),
        mesh=mesh,
        scratch_types=[
            [pltpu.VMEM((128,), jnp.int32)] * NBUF,
            pltpu.VMEM((8, DIM), jnp.float32),
            [pltpu.VMEM((128, DIM), jnp.float32)] * NBUF,
            [pltpu.VMEM((128, DIM), jnp.float32)] * NBUF,
            [pltpu.SemaphoreType.DMA] * NBUF,
            [pltpu.SemaphoreType.DMA] * NBUF,
        ],
        compiler_params=pltpu.CompilerParams(use_tc_tiling_on_sc=False,
                                             needs_layout_passes=False),
    )(_sc_body)
    out4 = f(idx4, tok2, pos)
    return out4.reshape(BATCH, SEQ, DIM)


def kernel(inputs, token_table, position_table):
    idx4 = (inputs.astype(jnp.int32)
            .reshape(BT, 128, SEQ // 8, 8)
            .transpose(2, 0, 3, 1)) * 2
    return _run(idx4, token_table.T, position_table)


# speedup vs baseline: 2.7156x; 2.3823x over previous
import functools

import jax
import jax.numpy as jnp
from jax import lax
from jax.experimental import pallas as pl
from jax.experimental.pallas import tpu as pltpu
from jax.experimental.pallas import tpu_sc as plsc

BATCH = 1024
SEQ = 200
VOCAB = 1000000
DIM = 64
LANES = 16
NUM_CORES = 2
NUM_SUBCORES = 16
NW = NUM_CORES * NUM_SUBCORES
BT = BATCH // 128
NUNITS = SEQ * BT
UNITS_PER_W = NUNITS // NW
NBUF = 5
NOUT = UNITS_PER_W // NBUF

TW = 4096
RGRID = (VOCAB + TW - 1) // TW


def _relayout_body(src_ref, dst_ref):
    dst_ref[:, 0:DIM] = jnp.transpose(src_ref[...], (1, 0))


def _sc_body(idx_hbm, tok_hbm, pos_hbm, out_hbm,
             ibufs, pos_v, gbufs, obufs, gsems, wsems):
    wid = lax.axis_index("s") * NUM_CORES + lax.axis_index("c")
    ubase = wid * UNITS_PER_W
    lmin = lax.min(ubase // BT, SEQ - 8)
    pltpu.sync_copy(pos_hbm.at[pl.ds(lmin, 8)], pos_v)

    def stage_and_fire(u, b):
        l = u // BT
        bt = u % BT
        pltpu.sync_copy(idx_hbm.at[l // 8, bt, l % 8], ibufs[b])
        return pltpu.async_copy(tok_hbm.at[ibufs[b]], gbufs[b], gsems[b])

    for b in range(NBUF):
        stage_and_fire(ubase + b, b)

    def outer(o, carry):
        for b in range(NBUF):
            u = ubase + o * NBUF + b
            l = u // BT
            bt = u % BT
            pltpu.make_async_copy(tok_hbm.at[ibufs[b]], gbufs[b],
                                  gsems[b]).wait()

            @pl.when(o > 0)
            def _drain():
                pltpu.make_async_copy(obufs[b], out_hbm.at[bt, :, l],
                                      wsems[b]).wait()

            pvecs = [pos_v[l - lmin, pl.ds(k * LANES, LANES)]
                     for k in range(DIM // LANES)]

            def tbody(t, c2, _b=b, _pv=pvecs):
                for k in range(DIM // LANES):
                    sl = pl.ds(k * LANES, LANES)
                    obufs[_b][t, sl] = gbufs[_b][t, sl] + _pv[k]
                return c2

            lax.fori_loop(0, 128, tbody, 0, unroll=4)

            pltpu.async_copy(obufs[b], out_hbm.at[bt, :, l], wsems[b])

            @pl.when(o < NOUT - 1)
            def _next():
                stage_and_fire(u + NBUF, b)
        return carry

    lax.fori_loop(0, NOUT, outer, 0)

    for b in range(NBUF):
        u = ubase + (NOUT - 1) * NBUF + b
        l = u // BT
        bt = u % BT
        pltpu.make_async_copy(obufs[b], out_hbm.at[bt, :, l],
                              wsems[b]).wait()


@jax.jit
def _run(idx4, tokT, pos):
    tokP = pl.pallas_call(
        _relayout_body,
        grid=(RGRID,),
        in_specs=[pl.BlockSpec((DIM, TW), lambda i: (0, i))],
        out_specs=pl.BlockSpec((TW, 128), lambda i: (i, 0)),
        out_shape=jax.ShapeDtypeStruct((VOCAB, 128), jnp.float32),
    )(tokT)
    tok2 = tokP.reshape(2 * VOCAB, DIM)

    mesh = plsc.VectorSubcoreMesh(core_axis_name="c", subcore_axis_name="s")
    f = functools.partial(
        pl.kernel,
        out_type=jax.ShapeDtypeStruct((BT, 128, SEQ, DIM), jnp.float32),
        mesh=mesh,
        scratch_types=[
            [pltpu.VMEM((128,), jnp.int32)] * NBUF,
            pltpu.VMEM((8, DIM), jnp.float32),
            [pltpu.VMEM((128, DIM), jnp.float32)] * NBUF,
            [pltpu.VMEM((128, DIM), jnp.float32)] * NBUF,
            [pltpu.SemaphoreType.DMA] * NBUF,
            [pltpu.SemaphoreType.DMA] * NBUF,
        ],
        compiler_params=pltpu.CompilerParams(use_tc_tiling_on_sc=False,
                                             needs_layout_passes=False),
    )(_sc_body)
    out4 = f(idx4, tok2, pos)
    return out4.reshape(BATCH, SEQ, DIM)


def kernel(inputs, token_table, position_table):
    idx4 = (inputs.astype(jnp.int32)
            .reshape(BT, 128, SEQ // 8, 8)
            .transpose(2, 0, 3, 1)) * 2
    return _run(idx4, token_table.T, position_table)


# speedup vs baseline: 3.1812x; 1.1715x over previous
import functools

import jax
import jax.numpy as jnp
from jax import lax
from jax.experimental import pallas as pl
from jax.experimental.pallas import tpu as pltpu
from jax.experimental.pallas import tpu_sc as plsc

BATCH = 1024
SEQ = 200
VOCAB = 1000000
DIM = 64
LANES = 16
NUM_CORES = 2
NUM_SUBCORES = 16
NW = NUM_CORES * NUM_SUBCORES
BT = BATCH // 128
NUNITS = SEQ * BT
UNITS_PER_W = NUNITS // NW
NBUF = 5
NOUT = UNITS_PER_W // NBUF

TW = 16384
RGRID = (VOCAB + TW - 1) // TW


def _relayout_body(src_ref, dst_ref):
    dst_ref[:, 0:DIM] = jnp.transpose(src_ref[...], (1, 0))


def _sc_body(idx_hbm, tok_hbm, pos_hbm, out_hbm,
             ibufs, pos_v, gbufs, obufs, gsems, wsems):
    wid = lax.axis_index("s") * NUM_CORES + lax.axis_index("c")
    ubase = wid * UNITS_PER_W
    lmin = lax.min(ubase // BT, SEQ - 8)
    pltpu.sync_copy(pos_hbm.at[pl.ds(lmin, 8)], pos_v)

    def stage_and_fire(u, b):
        l = u // BT
        bt = u % BT
        pltpu.sync_copy(idx_hbm.at[l // 8, bt, l % 8], ibufs[b])
        return pltpu.async_copy(tok_hbm.at[ibufs[b]], gbufs[b], gsems[b])

    for b in range(NBUF):
        stage_and_fire(ubase + b, b)

    def outer(o, carry):
        for b in range(NBUF):
            u = ubase + o * NBUF + b
            l = u // BT
            bt = u % BT
            pltpu.make_async_copy(tok_hbm.at[ibufs[b]], gbufs[b],
                                  gsems[b]).wait()

            @pl.when(o > 0)
            def _drain():
                pltpu.make_async_copy(obufs[b], out_hbm.at[bt, :, l],
                                      wsems[b]).wait()

            pvecs = [pos_v[l - lmin, pl.ds(k * LANES, LANES)]
                     for k in range(DIM // LANES)]

            def tbody(t, c2, _b=b, _pv=pvecs):
                for k in range(DIM // LANES):
                    sl = pl.ds(k * LANES, LANES)
                    obufs[_b][t, sl] = gbufs[_b][t, sl] + _pv[k]
                return c2

            lax.fori_loop(0, 128, tbody, 0, unroll=4)

            pltpu.async_copy(obufs[b], out_hbm.at[bt, :, l], wsems[b])

            @pl.when(o < NOUT - 1)
            def _next():
                stage_and_fire(u + NBUF, b)
        return carry

    lax.fori_loop(0, NOUT, outer, 0)

    for b in range(NBUF):
        u = ubase + (NOUT - 1) * NBUF + b
        l = u // BT
        bt = u % BT
        pltpu.make_async_copy(obufs[b], out_hbm.at[bt, :, l],
                              wsems[b]).wait()


@jax.jit
def _run(idx4, tokT, pos):
    tokP = pl.pallas_call(
        _relayout_body,
        grid=(RGRID,),
        in_specs=[pl.BlockSpec((DIM, TW), lambda i: (0, i))],
        out_specs=pl.BlockSpec((TW, 128), lambda i: (i, 0)),
        out_shape=jax.ShapeDtypeStruct((VOCAB, 128), jnp.float32),
    )(tokT)
    tok2 = tokP.reshape(2 * VOCAB, DIM)

    mesh = plsc.VectorSubcoreMesh(core_axis_name="c", subcore_axis_name="s")
    f = functools.partial(
        pl.kernel,
        out_type=jax.ShapeDtypeStruct((BT, 128, SEQ, DIM), jnp.float32),
        mesh=mesh,
        scratch_types=[
            [pltpu.VMEM((128,), jnp.int32)] * NBUF,
            pltpu.VMEM((8, DIM), jnp.float32),
            [pltpu.VMEM((128, DIM), jnp.float32)] * NBUF,
            [pltpu.VMEM((128, DIM), jnp.float32)] * NBUF,
            [pltpu.SemaphoreType.DMA] * NBUF,
            [pltpu.SemaphoreType.DMA] * NBUF,
        ],
        compiler_params=pltpu.CompilerParams(use_tc_tiling_on_sc=False,
                                             needs_layout_passes=False),
    )(_sc_body)
    out4 = f(idx4, tok2, pos)
    return out4.reshape(BATCH, SEQ, DIM)


def kernel(inputs, token_table, position_table):
    idx4 = (inputs.astype(jnp.int32)
            .reshape(BT, 128, SEQ // 8, 8)
            .transpose(2, 0, 3, 1)) * 2
    return _run(idx4, token_table.T, position_table)


# speedup vs baseline: 3.2251x; 1.0138x over previous
import functools

import jax
import jax.numpy as jnp
from jax import lax
from jax.experimental import pallas as pl
from jax.experimental.pallas import tpu as pltpu
from jax.experimental.pallas import tpu_sc as plsc

BATCH = 1024
SEQ = 200
VOCAB = 1000000
DIM = 64
LANES = 16
NUM_CORES = 2
NUM_SUBCORES = 16
NW = NUM_CORES * NUM_SUBCORES
BT = BATCH // 128
NUNITS = SEQ * BT
UNITS_PER_W = NUNITS // NW
NBUF = 5
NOUT = UNITS_PER_W // NBUF

TW = 32768
RGRID = (VOCAB + TW - 1) // TW


def _relayout_body(src_ref, dst_ref):
    dst_ref[:, 0:DIM] = jnp.transpose(src_ref[...], (1, 0))


def _sc_body(idx_hbm, tok_hbm, pos_hbm, out_hbm,
             ibufs, pos_v, gbufs, obufs, gsems, wsems):
    wid = lax.axis_index("s") * NUM_CORES + lax.axis_index("c")
    ubase = wid * UNITS_PER_W
    lmin = lax.min(ubase // BT, SEQ - 8)
    pltpu.sync_copy(pos_hbm.at[pl.ds(lmin, 8)], pos_v)

    def stage_and_fire(u, b):
        l = u // BT
        bt = u % BT
        pltpu.sync_copy(idx_hbm.at[l // 8, bt, l % 8], ibufs[b])
        return pltpu.async_copy(tok_hbm.at[ibufs[b]], gbufs[b], gsems[b])

    for b in range(NBUF):
        stage_and_fire(ubase + b, b)

    def outer(o, carry):
        for b in range(NBUF):
            u = ubase + o * NBUF + b
            l = u // BT
            bt = u % BT
            pltpu.make_async_copy(tok_hbm.at[ibufs[b]], gbufs[b],
                                  gsems[b]).wait()

            @pl.when(o > 0)
            def _drain():
                pltpu.make_async_copy(obufs[b], out_hbm.at[bt, :, l],
                                      wsems[b]).wait()

            pvecs = [pos_v[l - lmin, pl.ds(k * LANES, LANES)]
                     for k in range(DIM // LANES)]

            def tbody(t, c2, _b=b, _pv=pvecs):
                for k in range(DIM // LANES):
                    sl = pl.ds(k * LANES, LANES)
                    obufs[_b][t, sl] = gbufs[_b][t, sl] + _pv[k]
                return c2

            lax.fori_loop(0, 128, tbody, 0, unroll=4)

            pltpu.async_copy(obufs[b], out_hbm.at[bt, :, l], wsems[b])

            @pl.when(o < NOUT - 1)
            def _next():
                stage_and_fire(u + NBUF, b)
        return carry

    lax.fori_loop(0, NOUT, outer, 0)

    for b in range(NBUF):
        u = ubase + (NOUT - 1) * NBUF + b
        l = u // BT
        bt = u % BT
        pltpu.make_async_copy(obufs[b], out_hbm.at[bt, :, l],
                              wsems[b]).wait()


@jax.jit
def _run(idx4, tokT, pos):
    tokP = pl.pallas_call(
        _relayout_body,
        grid=(RGRID,),
        in_specs=[pl.BlockSpec((DIM, TW), lambda i: (0, i))],
        out_specs=pl.BlockSpec((TW, 128), lambda i: (i, 0)),
        out_shape=jax.ShapeDtypeStruct((VOCAB, 128), jnp.float32),
    )(tokT)
    tok2 = tokP.reshape(2 * VOCAB, DIM)

    mesh = plsc.VectorSubcoreMesh(core_axis_name="c", subcore_axis_name="s")
    f = functools.partial(
        pl.kernel,
        out_type=jax.ShapeDtypeStruct((BT, 128, SEQ, DIM), jnp.float32),
        mesh=mesh,
        scratch_types=[
            [pltpu.VMEM((128,), jnp.int32)] * NBUF,
            pltpu.VMEM((8, DIM), jnp.float32),
            [pltpu.VMEM((128, DIM), jnp.float32)] * NBUF,
            [pltpu.VMEM((128, DIM), jnp.float32)] * NBUF,
            [pltpu.SemaphoreType.DMA] * NBUF,
            [pltpu.SemaphoreType.DMA] * NBUF,
        ],
        compiler_params=pltpu.CompilerParams(use_tc_tiling_on_sc=False,
                                             needs_layout_passes=False),
    )(_sc_body)
    out4 = f(idx4, tok2, pos)
    return out4.reshape(BATCH, SEQ, DIM)


def kernel(inputs, token_table, position_table):
    idx4 = (inputs.astype(jnp.int32)
            .reshape(BT, 128, SEQ // 8, 8)
            .transpose(2, 0, 3, 1)) * 2
    return _run(idx4, token_table.T, position_table)
